# baseline (speedup 1.0000x reference)
import jax
import jax.numpy as jnp
from jax.experimental import pallas as pl
from jax.experimental.pallas import tpu as pltpu


def _round_up(x: int, m: int) -> int:
    return (x + m - 1) // m * m


def _fused_mlp_ce_kernel(x_ref, w1_ref, b1_ref, w2_ref, b2_ref, lbl_ref,
                         logits_ref, rowloss_ref):
    h = jnp.dot(x_ref[...], w1_ref[...], preferred_element_type=jnp.float32)
    h = jnp.maximum(h + b1_ref[...], 0.0)
    logits = jnp.dot(h.astype(jnp.bfloat16), w2_ref[...],
                     preferred_element_type=jnp.float32) + b2_ref[...]
    logits_ref[...] = logits

    lbl = lbl_ref[...]
    col = jax.lax.broadcasted_iota(jnp.int32, logits.shape, 1)
    m = jnp.max(logits, axis=-1, keepdims=True)
    lse = m + jnp.log(jnp.sum(jnp.exp(logits - m), axis=-1, keepdims=True))
    picked = jnp.sum(jnp.where(col == lbl, logits, 0.0), axis=-1,
                     keepdims=True)
    valid = (lbl >= 0).astype(jnp.float32)
    rowloss_ref[...] = (lse - picked) * valid


def kernel(x, labels, w1, b1, w2, b2):
    B, D = x.shape
    H = w1.shape[1]
    C = w2.shape[1]

    TB = min(512, _round_up(B, 8))
    nb = pl.cdiv(B, TB)
    Bp = nb * TB

    xb = x.astype(jnp.bfloat16)
    if Bp != B:
        xb = jnp.zeros((Bp, D), jnp.bfloat16).at[:B].set(xb)
        lbl = jnp.full((Bp, 1), -1, jnp.int32).at[:B, 0].set(
            labels.astype(jnp.int32))
    else:
        lbl = labels.astype(jnp.int32).reshape(B, 1)
    w1b = w1.astype(jnp.bfloat16)
    w2b = w2.astype(jnp.bfloat16)
    b1r = b1.reshape(1, H)
    b2r = b2.reshape(1, C)

    logits_pad, row_loss = pl.pallas_call(
        _fused_mlp_ce_kernel,
        out_shape=(jax.ShapeDtypeStruct((Bp, C), jnp.float32),
                   jax.ShapeDtypeStruct((Bp, 1), jnp.float32)),
        grid=(nb,),
        in_specs=[
            pl.BlockSpec((TB, D), lambda i: (i, 0)),
            pl.BlockSpec((D, H), lambda i: (0, 0)),
            pl.BlockSpec((1, H), lambda i: (0, 0)),
            pl.BlockSpec((H, C), lambda i: (0, 0)),
            pl.BlockSpec((1, C), lambda i: (0, 0)),
            pl.BlockSpec((TB, 1), lambda i: (i, 0)),
        ],
        out_specs=(pl.BlockSpec((TB, C), lambda i: (i, 0)),
                   pl.BlockSpec((TB, 1), lambda i: (i, 0))),
        compiler_params=pltpu.CompilerParams(
            dimension_semantics=("parallel",)),
    )(xb, w1b, b1r, w2b, b2r, lbl)

    logits = logits_pad if Bp == B else logits_pad[:B]
    loss = jnp.sum(row_loss) / B
    return logits, loss


# speedup vs baseline: 1.4118x; 1.4118x over previous
import jax
import jax.numpy as jnp
from jax.experimental import pallas as pl
from jax.experimental.pallas import tpu as pltpu


def _round_up(x: int, m: int) -> int:
    return (x + m - 1) // m * m


def _fused_mlp_ce_kernel(x_ref, w1_ref, b1_ref, w2_ref, b2_ref, lbl_ref,
                         logits_ref, lpart_ref):
    h = jnp.dot(x_ref[...], w1_ref[...], preferred_element_type=jnp.float32)
    h = jnp.maximum(h + b1_ref[...], 0.0)
    logits = jnp.dot(h, w2_ref[...],
                     preferred_element_type=jnp.float32) + b2_ref[...]
    logits_ref[...] = logits

    lbl = lbl_ref[...]
    col = jax.lax.broadcasted_iota(jnp.int32, logits.shape, 1)
    m = jnp.max(logits, axis=-1, keepdims=True)
    lse = m + jnp.log(jnp.sum(jnp.exp(logits - m), axis=-1, keepdims=True))
    picked = jnp.sum(jnp.where(col == lbl, logits, 0.0), axis=-1,
                     keepdims=True)
    valid = (lbl >= 0).astype(jnp.float32)
    lpart_ref[...] = jnp.sum((lse - picked) * valid).reshape(1, 1, 1)


def kernel(x, labels, w1, b1, w2, b2):
    B, D = x.shape
    H = w1.shape[1]
    C = w2.shape[1]

    TB = min(1024, _round_up(B, 8))
    nb = pl.cdiv(B, TB)
    Bp = nb * TB

    if Bp != B:
        xp = jnp.zeros((Bp, D), x.dtype).at[:B].set(x)
        lbl = jnp.full((Bp, 1), -1, jnp.int32).at[:B, 0].set(
            labels.astype(jnp.int32))
    else:
        xp = x
        lbl = labels.astype(jnp.int32).reshape(B, 1)
    b1r = b1.reshape(1, H)
    b2r = b2.reshape(1, C)

    logits_pad, lparts = pl.pallas_call(
        _fused_mlp_ce_kernel,
        out_shape=(jax.ShapeDtypeStruct((Bp, C), jnp.float32),
                   jax.ShapeDtypeStruct((nb, 1, 1), jnp.float32)),
        grid=(nb,),
        in_specs=[
            pl.BlockSpec((TB, D), lambda i: (i, 0)),
            pl.BlockSpec((D, H), lambda i: (0, 0)),
            pl.BlockSpec((1, H), lambda i: (0, 0)),
            pl.BlockSpec((H, C), lambda i: (0, 0)),
            pl.BlockSpec((1, C), lambda i: (0, 0)),
            pl.BlockSpec((TB, 1), lambda i: (i, 0)),
        ],
        out_specs=(pl.BlockSpec((TB, C), lambda i: (i, 0)),
                   pl.BlockSpec((1, 1, 1), lambda i: (i, 0, 0))),
        compiler_params=pltpu.CompilerParams(
            dimension_semantics=("parallel",)),
    )(xp, w1, b1r, w2, b2r, lbl)

    logits = logits_pad if Bp == B else logits_pad[:B]
    loss = jnp.sum(lparts) / B
    return logits, loss
